# baseline (speedup 1.0000x reference)
import functools

import jax
import jax.numpy as jnp
from jax import lax
from jax.experimental import pallas as pl
from jax.experimental.pallas import tpu as pltpu
from jax.experimental.pallas import tpu_sc as plsc

_NUM_CORES = 2
_NUM_SUBCORES = 16
_LANES = 16
_NW = _NUM_CORES * _NUM_SUBCORES

_CHUNK = 128
_GROUP = 8


@functools.lru_cache(maxsize=None)
def _build(total, vocab_rows, num_fields, dim):
    per_w = total // _NW
    n_chunks = per_w // _CHUNK
    n_groups = n_chunks // _GROUP
    group_rows = _GROUP * _CHUNK
    mesh = plsc.VectorSubcoreMesh(core_axis_name="c", subcore_axis_name="s")

    @functools.partial(
        pl.kernel,
        out_type=jax.ShapeDtypeStruct((total, dim), jnp.float32),
        mesh=mesh,
        compiler_params=pltpu.CompilerParams(use_tc_tiling_on_sc=False),
        scratch_types=[
            pltpu.MemorySpace.VMEM((per_w,), jnp.int32),
            pltpu.MemorySpace.VMEM((n_chunks, _CHUNK), jnp.int32),
            pltpu.MemorySpace.VMEM((group_rows, dim), jnp.float32),
            pltpu.SemaphoreType.DMA,
        ],
    )
    def run(idx_hbm, tab_hbm, out_hbm, raw_v, idx2_v, rows_v, sem):
        wid = lax.axis_index("s") * _NUM_CORES + lax.axis_index("c")
        base = wid * per_w

        pltpu.sync_copy(idx_hbm.at[pl.ds(base, per_w)], raw_v)

        lane = lax.iota(jnp.int32, _LANES)

        def idx_body(c, _):
            for k in range(_CHUNK // _LANES):
                off = c * _CHUNK + k * _LANES
                raw = raw_v[pl.ds(off, _LANES)]
                pos = base + off + lane
                field = lax.rem(pos, num_fields)
                idx2_v[c, pl.ds(k * _LANES, _LANES)] = raw + field * vocab_rows
            return 0

        lax.fori_loop(0, n_chunks, idx_body, 0)

        def g_body(g, _):
            copies = []
            for k in range(_GROUP):
                copies.append(
                    pltpu.async_copy(
                        tab_hbm.at[idx2_v.at[g * _GROUP + k]],
                        rows_v.at[pl.ds(k * _CHUNK, _CHUNK)],
                        sem,
                    )
                )
            for cp in copies:
                cp.wait()
            pltpu.sync_copy(
                rows_v, out_hbm.at[pl.ds(base + g * group_rows, group_rows)]
            )
            return 0

        lax.fori_loop(0, n_groups, g_body, 0)

    return run


def kernel(inputs, tables):
    batch, num_fields = inputs.shape
    _, vocab_rows, dim = tables.shape
    total = batch * num_fields
    idx_flat = inputs.reshape(total).astype(jnp.int32)
    tab_flat = tables.reshape(num_fields * vocab_rows, dim)
    out = _build(total, vocab_rows, num_fields, dim)(idx_flat, tab_flat)
    return out.reshape(batch, num_fields, dim)


# speedup vs baseline: 2.9179x; 2.9179x over previous
import functools

import jax
import jax.numpy as jnp
from jax import lax
from jax.experimental import pallas as pl
from jax.experimental.pallas import tpu as pltpu
from jax.experimental.pallas import tpu_sc as plsc

_NUM_CORES = 2
_NUM_SUBCORES = 16
_LANES = 16
_NW = _NUM_CORES * _NUM_SUBCORES

_OUT_CHUNK = 8192


@functools.lru_cache(maxsize=None)
def _build(num_fields, vocab_rows, dim, batch):
    n_pairs = num_fields * dim
    per_w = n_pairs // _NW
    vpad = (vocab_rows + 7) // 8 * 8
    n_chunks = batch // _OUT_CHUNK
    mesh = plsc.VectorSubcoreMesh(core_axis_name="c", subcore_axis_name="s")

    @functools.partial(
        pl.kernel,
        out_type=jax.ShapeDtypeStruct((num_fields, dim, batch), jnp.float32),
        mesh=mesh,
        compiler_params=pltpu.CompilerParams(
            use_tc_tiling_on_sc=False, needs_layout_passes=False
        ),
        scratch_types=[
            pltpu.MemorySpace.VMEM((vpad,), jnp.float32),
            pltpu.MemorySpace.VMEM((batch,), jnp.int32),
            pltpu.MemorySpace.VMEM((_OUT_CHUNK,), jnp.float32),
        ],
    )
    def run(idx_hbm, tab_hbm, out_hbm, tab_v, idx_v, out_v):
        wid = lax.axis_index("s") * _NUM_CORES + lax.axis_index("c")
        p0 = wid * per_w

        def pair_body(p, prev_f):
            f = p // dim
            c = lax.rem(p, dim)

            @pl.when(jnp.logical_or(p == p0, f != prev_f))
            def _():
                pltpu.sync_copy(idx_hbm.at[f], idx_v)

            pltpu.sync_copy(tab_hbm.at[f, c], tab_v.at[pl.ds(0, vocab_rows)])

            def chunk_body(h, _):
                base = h * _OUT_CHUNK

                def vec_body(j, _):
                    off = j * _LANES
                    iv = idx_v[pl.ds(base + off, _LANES)]
                    out_v[pl.ds(off, _LANES)] = plsc.load_gather(tab_v, [iv])
                    return 0

                lax.fori_loop(0, _OUT_CHUNK // _LANES, vec_body, 0, unroll=8)
                pltpu.sync_copy(
                    out_v, out_hbm.at[f, c, pl.ds(base, _OUT_CHUNK)]
                )
                return 0

            lax.fori_loop(0, n_chunks, chunk_body, 0)
            return f

        lax.fori_loop(p0, p0 + per_w, pair_body, -1)

    return run


def kernel(inputs, tables):
    batch, num_fields = inputs.shape
    _, vocab_rows, dim = tables.shape
    idx_t = inputs.T.astype(jnp.int32)
    tab_t = jnp.transpose(tables, (0, 2, 1))
    out_t = _build(num_fields, vocab_rows, dim, batch)(idx_t, tab_t)
    return jnp.transpose(out_t, (2, 0, 1))


# speedup vs baseline: 32.2208x; 11.0424x over previous
import functools

import jax
import jax.numpy as jnp
from jax import lax
from jax.experimental import pallas as pl
from jax.experimental.pallas import tpu as pltpu
from jax.experimental.pallas import tpu_sc as plsc

_NUM_CORES = 2
_NUM_SUBCORES = 16
_LANES = 16
_NW = _NUM_CORES * _NUM_SUBCORES

_OUT_CHUNK = 8192


@functools.lru_cache(maxsize=None)
def _build(num_fields, vocab_rows, dim, batch):
    n_rows = num_fields * dim
    per_w = n_rows // _NW
    n_chunks = batch // _OUT_CHUNK
    mesh = plsc.VectorSubcoreMesh(core_axis_name="c", subcore_axis_name="s")

    @functools.partial(
        pl.kernel,
        out_type=jax.ShapeDtypeStruct((n_rows, batch), jnp.float32),
        mesh=mesh,
        compiler_params=pltpu.CompilerParams(
            use_tc_tiling_on_sc=True, needs_layout_passes=False
        ),
        scratch_types=[
            pltpu.MemorySpace.VMEM((vocab_rows,), jnp.float32),
            pltpu.MemorySpace.VMEM((batch,), jnp.int32),
            pltpu.MemorySpace.VMEM((_OUT_CHUNK,), jnp.float32),
        ],
    )
    def run(idx_hbm, tab_hbm, out_hbm, tab_v, idx_v, out_v):
        wid = lax.axis_index("s") * _NUM_CORES + lax.axis_index("c")
        r0 = wid * per_w

        def task_body(r, prev_f):
            f = r // dim

            @pl.when(f != prev_f)
            def _():
                pltpu.sync_copy(idx_hbm.at[f], idx_v)

            pltpu.sync_copy(tab_hbm.at[r], tab_v)

            def chunk_body(h, _):
                base = h * _OUT_CHUNK

                def vec_body(j, _):
                    off = j * _LANES
                    iv = idx_v[pl.ds(base + off, _LANES)]
                    out_v[pl.ds(off, _LANES)] = plsc.load_gather(tab_v, [iv])
                    return 0

                lax.fori_loop(0, _OUT_CHUNK // _LANES, vec_body, 0, unroll=8)
                pltpu.sync_copy(
                    out_v, out_hbm.at[r, pl.ds(base, _OUT_CHUNK)]
                )
                return 0

            lax.fori_loop(0, n_chunks, chunk_body, 0)
            return f

        lax.fori_loop(r0, r0 + per_w, task_body, -1)

    return run


def kernel(inputs, tables):
    batch, num_fields = inputs.shape
    _, vocab_rows, dim = tables.shape
    idx_t = inputs.T.astype(jnp.int32)
    tab2 = jnp.transpose(tables, (0, 2, 1)).reshape(
        num_fields * dim, vocab_rows
    )
    out2 = _build(num_fields, vocab_rows, dim, batch)(idx_t, tab2)
    out3 = out2.reshape(num_fields, dim, batch)
    return jnp.transpose(out3, (2, 0, 1))


# speedup vs baseline: 32.3261x; 1.0033x over previous
import functools

import jax
import jax.numpy as jnp
from jax import lax
from jax.experimental import pallas as pl
from jax.experimental.pallas import tpu as pltpu
from jax.experimental.pallas import tpu_sc as plsc

_NUM_CORES = 2
_NUM_SUBCORES = 16
_LANES = 16
_NW = _NUM_CORES * _NUM_SUBCORES

_OUT_CHUNK = 8192


@functools.lru_cache(maxsize=None)
def _build(num_fields, vocab_rows, dim, batch):
    n_rows = num_fields * dim
    per_w = n_rows // _NW
    n_chunks = batch // _OUT_CHUNK
    mesh = plsc.VectorSubcoreMesh(core_axis_name="c", subcore_axis_name="s")

    @functools.partial(
        pl.kernel,
        out_type=jax.ShapeDtypeStruct((n_rows, batch), jnp.float32),
        mesh=mesh,
        compiler_params=pltpu.CompilerParams(
            use_tc_tiling_on_sc=True, needs_layout_passes=False
        ),
        scratch_types=[
            pltpu.MemorySpace.VMEM((vocab_rows,), jnp.float32),
            pltpu.MemorySpace.VMEM((batch,), jnp.int32),
            pltpu.MemorySpace.VMEM((_OUT_CHUNK,), jnp.float32),
        ],
    )
    def run(idx_hbm, tab_hbm, out_hbm, tab_v, idx_v, out_v):
        wid = lax.axis_index("s") * _NUM_CORES + lax.axis_index("c")
        r0 = wid * per_w

        def task_body(r, prev_f):
            f = r // dim

            @pl.when(f != prev_f)
            def _():
                pltpu.sync_copy(idx_hbm.at[f], idx_v)

            pltpu.sync_copy(tab_hbm.at[r], tab_v)

            for h in range(n_chunks):
                base = h * _OUT_CHUNK

                def vec_body(j, _, base=base):
                    off = j * _LANES
                    iv = idx_v[pl.ds(base + off, _LANES)]
                    out_v[pl.ds(off, _LANES)] = plsc.load_gather(tab_v, [iv])
                    return 0

                lax.fori_loop(0, _OUT_CHUNK // _LANES, vec_body, 0, unroll=16)
                pltpu.sync_copy(
                    out_v, out_hbm.at[r, pl.ds(base, _OUT_CHUNK)]
                )
            return f

        lax.fori_loop(r0, r0 + per_w, task_body, -1)

    return run


def kernel(inputs, tables):
    batch, num_fields = inputs.shape
    _, vocab_rows, dim = tables.shape
    idx_t = inputs.T.astype(jnp.int32)
    tab2 = jnp.transpose(tables, (0, 2, 1)).reshape(
        num_fields * dim, vocab_rows
    )
    out2 = _build(num_fields, vocab_rows, dim, batch)(idx_t, tab2)
    out3 = out2.reshape(num_fields, dim, batch)
    return jnp.transpose(out3, (2, 0, 1))


# speedup vs baseline: 32.4439x; 1.0036x over previous
import functools

import jax
import jax.numpy as jnp
from jax import lax
from jax.experimental import pallas as pl
from jax.experimental.pallas import tpu as pltpu
from jax.experimental.pallas import tpu_sc as plsc

_NUM_CORES = 2
_NUM_SUBCORES = 16
_LANES = 16
_NW = _NUM_CORES * _NUM_SUBCORES

_OUT_CHUNK = 4096
_NBUF = 2


@functools.lru_cache(maxsize=None)
def _build(num_fields, vocab_rows, dim, batch):
    n_rows = num_fields * dim
    per_w = n_rows // _NW
    n_chunks = batch // _OUT_CHUNK
    mesh = plsc.VectorSubcoreMesh(core_axis_name="c", subcore_axis_name="s")

    @functools.partial(
        pl.kernel,
        out_type=jax.ShapeDtypeStruct((n_rows, batch), jnp.float32),
        mesh=mesh,
        compiler_params=pltpu.CompilerParams(
            use_tc_tiling_on_sc=True, needs_layout_passes=False
        ),
        scratch_types=[
            pltpu.MemorySpace.VMEM((vocab_rows,), jnp.float32),
            pltpu.MemorySpace.VMEM((batch,), jnp.int32),
            pltpu.MemorySpace.VMEM((_NBUF, _OUT_CHUNK), jnp.float32),
            pltpu.SemaphoreType.DMA,
        ],
    )
    def run(idx_hbm, tab_hbm, out_hbm, tab_v, idx_v, out_v, sem):
        wid = lax.axis_index("s") * _NUM_CORES + lax.axis_index("c")
        r0 = wid * per_w

        def task_body(i, prev_f):
            r = r0 + i
            f = r // dim

            @pl.when(f != prev_f)
            def _():
                pltpu.sync_copy(idx_hbm.at[f], idx_v)

            pltpu.sync_copy(tab_hbm.at[r], tab_v)

            for h in range(n_chunks):
                b = h % _NBUF
                base = h * _OUT_CHUNK

                def drain():
                    pltpu.make_async_copy(
                        out_v.at[0], out_hbm.at[r0, pl.ds(0, _OUT_CHUNK)], sem
                    ).wait()

                if h < _NBUF:
                    @pl.when(i > 0)
                    def _():
                        drain()
                else:
                    drain()

                def vec_body(j, _, base=base, b=b):
                    off = j * _LANES
                    iv = idx_v[pl.ds(base + off, _LANES)]
                    out_v[b, pl.ds(off, _LANES)] = plsc.load_gather(tab_v, [iv])
                    return 0

                lax.fori_loop(0, _OUT_CHUNK // _LANES, vec_body, 0, unroll=16)
                pltpu.async_copy(
                    out_v.at[b], out_hbm.at[r, pl.ds(base, _OUT_CHUNK)], sem
                )
            return f

        lax.fori_loop(0, per_w, task_body, -1)

        for _ in range(min(_NBUF, n_chunks)):
            pltpu.make_async_copy(
                out_v.at[0], out_hbm.at[r0, pl.ds(0, _OUT_CHUNK)], sem
            ).wait()

    return run


def kernel(inputs, tables):
    batch, num_fields = inputs.shape
    _, vocab_rows, dim = tables.shape
    idx_t = inputs.T.astype(jnp.int32)
    tab2 = jnp.transpose(tables, (0, 2, 1)).reshape(
        num_fields * dim, vocab_rows
    )
    out2 = _build(num_fields, vocab_rows, dim, batch)(idx_t, tab2)
    out3 = out2.reshape(num_fields, dim, batch)
    return jnp.transpose(out3, (2, 0, 1))


# speedup vs baseline: 64.3740x; 1.9842x over previous
import functools

import jax
import jax.numpy as jnp
from jax import lax
from jax.experimental import pallas as pl
from jax.experimental.pallas import tpu as pltpu
from jax.experimental.pallas import tpu_sc as plsc

_NUM_CORES = 2
_NUM_SUBCORES = 16
_LANES = 16
_NW = _NUM_CORES * _NUM_SUBCORES

_OUT_CHUNK = 4096
_NBUF = 2


@functools.lru_cache(maxsize=None)
def _build(num_fields, vocab_rows, dim, batch):
    n_rows = num_fields * dim
    per_w = n_rows // _NW
    n_chunks = batch // _OUT_CHUNK
    mesh = plsc.VectorSubcoreMesh(core_axis_name="c", subcore_axis_name="s")

    @functools.partial(
        pl.kernel,
        out_type=jax.ShapeDtypeStruct((n_rows, batch), jnp.float32),
        mesh=mesh,
        compiler_params=pltpu.CompilerParams(
            use_tc_tiling_on_sc=True, needs_layout_passes=False
        ),
        scratch_types=[
            pltpu.MemorySpace.VMEM((vocab_rows,), jnp.float32),
            pltpu.MemorySpace.VMEM((batch,), jnp.int32),
            pltpu.MemorySpace.VMEM((_NBUF, _OUT_CHUNK), jnp.float32),
            pltpu.SemaphoreType.DMA,
        ],
    )
    def run(idx_hbm, tab_hbm, out_hbm, tab_v, idx_v, out_v, sem):
        wid = lax.axis_index("s") * _NUM_CORES + lax.axis_index("c")
        r0 = wid * per_w

        def task_body(i, prev_f):
            r = r0 + i
            f = r // dim

            @pl.when(f != prev_f)
            def _():
                pltpu.sync_copy(idx_hbm.at[f], idx_v)

            pltpu.sync_copy(tab_hbm.at[r], tab_v)

            for h in range(n_chunks):
                b = h % _NBUF
                base = h * _OUT_CHUNK

                def drain():
                    pltpu.make_async_copy(
                        out_v.at[0], out_hbm.at[r0, pl.ds(0, _OUT_CHUNK)], sem
                    ).wait()

                if h < _NBUF:
                    @pl.when(i > 0)
                    def _():
                        drain()
                else:
                    drain()

                def vec_body(j, base=base, b=b):
                    off = j * _LANES
                    iv = idx_v[pl.ds(base + off, _LANES)]
                    out_v[b, pl.ds(off, _LANES)] = plsc.load_gather(tab_v, [iv])

                plsc.parallel_loop(0, _OUT_CHUNK // _LANES, unroll=8)(vec_body)
                pltpu.async_copy(
                    out_v.at[b], out_hbm.at[r, pl.ds(base, _OUT_CHUNK)], sem
                )
            return f

        lax.fori_loop(0, per_w, task_body, -1)

        for _ in range(min(_NBUF, n_chunks)):
            pltpu.make_async_copy(
                out_v.at[0], out_hbm.at[r0, pl.ds(0, _OUT_CHUNK)], sem
            ).wait()

    return run


def kernel(inputs, tables):
    batch, num_fields = inputs.shape
    _, vocab_rows, dim = tables.shape
    idx_t = inputs.T.astype(jnp.int32)
    tab2 = jnp.transpose(tables, (0, 2, 1)).reshape(
        num_fields * dim, vocab_rows
    )
    out2 = _build(num_fields, vocab_rows, dim, batch)(idx_t, tab2)
    out3 = out2.reshape(num_fields, dim, batch)
    return jnp.transpose(out3, (2, 0, 1))


# speedup vs baseline: 64.4283x; 1.0008x over previous
import functools

import jax
import jax.numpy as jnp
from jax import lax
from jax.experimental import pallas as pl
from jax.experimental.pallas import tpu as pltpu
from jax.experimental.pallas import tpu_sc as plsc

_NUM_CORES = 2
_NUM_SUBCORES = 16
_LANES = 16
_NW = _NUM_CORES * _NUM_SUBCORES

_OUT_CHUNK = 4096
_NBUF = 2


@functools.lru_cache(maxsize=None)
def _build(num_fields, vocab_rows, dim, batch):
    n_rows = num_fields * dim
    per_w = n_rows // _NW
    n_chunks = batch // _OUT_CHUNK
    mesh = plsc.VectorSubcoreMesh(core_axis_name="c", subcore_axis_name="s")

    @functools.partial(
        pl.kernel,
        out_type=jax.ShapeDtypeStruct((n_rows, batch), jnp.float32),
        mesh=mesh,
        compiler_params=pltpu.CompilerParams(
            use_tc_tiling_on_sc=True, needs_layout_passes=False
        ),
        scratch_types=[
            pltpu.MemorySpace.VMEM((vocab_rows,), jnp.float32),
            pltpu.MemorySpace.VMEM((batch,), jnp.int32),
            pltpu.MemorySpace.VMEM((_NBUF, _OUT_CHUNK), jnp.float32),
            pltpu.SemaphoreType.DMA,
        ],
    )
    def run(idx_hbm, tab_hbm, out_hbm, tab_v, idx_v, out_v, sem):
        wid = lax.axis_index("s") * _NUM_CORES + lax.axis_index("c")
        r0 = wid * per_w

        def task_body(i, prev_f):
            r = r0 + i
            f = r // dim

            @pl.when(f != prev_f)
            def _():
                pltpu.sync_copy(idx_hbm.at[f], idx_v)

            pltpu.sync_copy(tab_hbm.at[r], tab_v)

            for h in range(n_chunks):
                b = h % _NBUF
                base = h * _OUT_CHUNK

                def drain():
                    pltpu.make_async_copy(
                        out_v.at[0], out_hbm.at[r0, pl.ds(0, _OUT_CHUNK)], sem
                    ).wait()

                if h < _NBUF:
                    @pl.when(i > 0)
                    def _():
                        drain()
                else:
                    drain()

                def vec_body(j, base=base, b=b):
                    off = j * _LANES
                    iv = idx_v[pl.ds(base + off, _LANES)]
                    out_v[b, pl.ds(off, _LANES)] = plsc.load_gather(tab_v, [iv])

                plsc.parallel_loop(0, _OUT_CHUNK // _LANES, unroll=16)(vec_body)
                pltpu.async_copy(
                    out_v.at[b], out_hbm.at[r, pl.ds(base, _OUT_CHUNK)], sem
                )
            return f

        lax.fori_loop(0, per_w, task_body, -1)

        for _ in range(min(_NBUF, n_chunks)):
            pltpu.make_async_copy(
                out_v.at[0], out_hbm.at[r0, pl.ds(0, _OUT_CHUNK)], sem
            ).wait()

    return run


def kernel(inputs, tables):
    batch, num_fields = inputs.shape
    _, vocab_rows, dim = tables.shape
    idx_t = inputs.T.astype(jnp.int32)
    tab2 = jnp.transpose(tables, (0, 2, 1)).reshape(
        num_fields * dim, vocab_rows
    )
    out2 = _build(num_fields, vocab_rows, dim, batch)(idx_t, tab2)
    out3 = out2.reshape(num_fields, dim, batch)
    return jnp.transpose(out3, (2, 0, 1))
